# baseline (speedup 1.0000x reference)
import jax
import jax.numpy as jnp
from jax.experimental import pallas as pl
from jax.experimental.pallas import tpu as pltpu

_B = 1024
_D = 16
_A = 100000

_BB = 64
_TA = 2048
_CH = 128
_NC = _TA // _CH
_NB = _B // _BB
_NA = (_A + _TA - 1) // _TA

_NEG = -1e30
_IMAX = 2**31 - 1


def _gfn_kernel(s_ref, w_ref, b_ref, c_ref, u_ref, act_ref, lp_ref,
                az_ref, ai_ref, al_ref, as_ref):
    a = pl.program_id(1)

    @pl.when(a == 0)
    def _init():
        az_ref[:] = jnp.full((_BB, _CH), _NEG, jnp.float32)
        ai_ref[:] = jnp.zeros((_BB, _CH), jnp.int32)
        al_ref[:] = jnp.zeros((_BB, _CH), jnp.float32)
        as_ref[:] = jnp.zeros((_BB, _CH), jnp.float32)

    logits = jnp.dot(s_ref[:], w_ref[:], preferred_element_type=jnp.float32)
    c = c_ref[0, 0, :][:, None]
    lane = jax.lax.broadcasted_iota(jnp.int32, (_BB, _CH), 1)

    def body(masked):
        acc_z = az_ref[:]
        acc_i = ai_ref[:]
        acc_l = al_ref[:]
        acc_s = as_ref[:]
        for ci in range(_NC):
            lgt = logits[:, ci * _CH:(ci + 1) * _CH] + b_ref[ci * _CH:(ci + 1) * _CH][None, :]
            u = u_ref[:, ci * _CH:(ci + 1) * _CH]
            col = lane + (a * _TA + ci * _CH)
            g = -jnp.log(-jnp.log(u))
            z = lgt + g
            if masked:
                valid = col < _A
                z = jnp.where(valid, z, _NEG)
                lgt = jnp.where(valid, lgt, _NEG)
            upd = z > acc_z
            acc_z = jnp.where(upd, z, acc_z)
            acc_i = jnp.where(upd, col, acc_i)
            acc_l = jnp.where(upd, lgt, acc_l)
            acc_s = acc_s + jnp.exp(lgt - c)
        az_ref[:] = acc_z
        ai_ref[:] = acc_i
        al_ref[:] = acc_l
        as_ref[:] = acc_s

    @pl.when(a < _NA - 1)
    def _full():
        body(False)

    @pl.when(a == _NA - 1)
    def _last():
        body(True)
        acc_z = az_ref[:]
        zrow = jnp.max(acc_z, axis=1)
        eq = acc_z == zrow[:, None]
        idxrow = jnp.min(jnp.where(eq, ai_ref[:], _IMAX), axis=1)
        lrow = jnp.sum(jnp.where(ai_ref[:] == idxrow[:, None], al_ref[:], 0.0),
                       axis=1)
        srow = jnp.sum(as_ref[:], axis=1)
        act_ref[0, 0, :] = idxrow
        lp_ref[0, 0, :] = lrow - (c_ref[0, 0, :] + jnp.log(srow))


def kernel(s, u, W, b):
    wn = jnp.sqrt(jnp.max(jnp.sum(W * W, axis=0)))
    sn = jnp.sqrt(jnp.sum(s * s, axis=1))
    c = (sn * wn + jnp.max(b) + 1.0).reshape(_NB, 1, _BB)

    actions, log_prob = pl.pallas_call(
        _gfn_kernel,
        grid=(_NB, _NA),
        in_specs=[
            pl.BlockSpec((_BB, _D), lambda i, j: (i, 0)),
            pl.BlockSpec((_D, _TA), lambda i, j: (0, j)),
            pl.BlockSpec((_TA,), lambda i, j: (j,)),
            pl.BlockSpec((1, 1, _BB), lambda i, j: (i, 0, 0)),
            pl.BlockSpec((_BB, _TA), lambda i, j: (i, j)),
        ],
        out_specs=[
            pl.BlockSpec((1, 1, _BB), lambda i, j: (i, 0, 0)),
            pl.BlockSpec((1, 1, _BB), lambda i, j: (i, 0, 0)),
        ],
        out_shape=[
            jax.ShapeDtypeStruct((_NB, 1, _BB), jnp.int32),
            jax.ShapeDtypeStruct((_NB, 1, _BB), jnp.float32),
        ],
        scratch_shapes=[
            pltpu.VMEM((_BB, _CH), jnp.float32),
            pltpu.VMEM((_BB, _CH), jnp.int32),
            pltpu.VMEM((_BB, _CH), jnp.float32),
            pltpu.VMEM((_BB, _CH), jnp.float32),
        ],
        compiler_params=pltpu.CompilerParams(
            dimension_semantics=("parallel", "arbitrary"),
        ),
    )(s, W, b, c, u)
    return (actions.reshape(_B), log_prob.reshape(_B))


# speedup vs baseline: 1.4886x; 1.4886x over previous
import jax
import jax.numpy as jnp
from jax.experimental import pallas as pl
from jax.experimental.pallas import tpu as pltpu

_B = 1024
_D = 16
_A = 100000

_BB = 64
_TA = 8192
_CH = 128
_NC = _TA // _CH
_NB = _B // _BB
_NA = (_A + _TA - 1) // _TA
_AP = _NA * _TA

_NEG = -1e30
_IMAX = 2**31 - 1


def _gfn_kernel(s_ref, w_ref, b_ref, c_ref, u_ref, act_ref, lp_ref,
                az_ref, ac_ref, al_ref, as_ref):
    a = pl.program_id(1)

    @pl.when(a == 0)
    def _init():
        az_ref[:] = jnp.full((_BB, _CH), _NEG, jnp.float32)
        ac_ref[:] = jnp.zeros((_BB, _CH), jnp.int32)
        al_ref[:] = jnp.zeros((_BB, _CH), jnp.float32)
        as_ref[:] = jnp.zeros((_BB, _CH), jnp.float32)

    logits = jnp.dot(s_ref[:], w_ref[:], preferred_element_type=jnp.float32)
    c = c_ref[0, 0, :][:, None]

    acc_z = az_ref[:]
    acc_c = ac_ref[:]
    acc_l = al_ref[:]
    acc_s = as_ref[:]
    for ci in range(_NC):
        sl = slice(ci * _CH, (ci + 1) * _CH)
        lgt = logits[:, sl] + b_ref[sl][None, :]
        g = -jnp.log(-jnp.log(u_ref[:, sl]))
        z = lgt + jnp.minimum(g, 100.0)
        upd = z > acc_z
        cid = jnp.full((_BB, _CH), a * _NC + ci, jnp.int32)
        acc_z = jnp.where(upd, z, acc_z)
        acc_c = jnp.where(upd, cid, acc_c)
        acc_l = jnp.where(upd, lgt, acc_l)
        acc_s = acc_s + jnp.exp(lgt - c)
    az_ref[:] = acc_z
    ac_ref[:] = acc_c
    al_ref[:] = acc_l
    as_ref[:] = acc_s

    @pl.when(a == _NA - 1)
    def _fin():
        lane = jax.lax.broadcasted_iota(jnp.int32, (_BB, _CH), 1)
        col = ac_ref[:] * _CH + lane
        acc_z = az_ref[:]
        zrow = jnp.max(acc_z, axis=1)
        eq = acc_z == zrow[:, None]
        idxrow = jnp.min(jnp.where(eq, col, _IMAX), axis=1)
        lrow = jnp.sum(jnp.where(col == idxrow[:, None], al_ref[:], 0.0),
                       axis=1)
        srow = jnp.sum(as_ref[:], axis=1)
        act_ref[0, 0, :] = idxrow
        lp_ref[0, 0, :] = lrow - (c_ref[0, 0, :] + jnp.log(srow))


def kernel(s, u, W, b):
    wn = jnp.sqrt(jnp.max(jnp.sum(W * W, axis=0)))
    sn = jnp.sqrt(jnp.sum(s * s, axis=1))
    c = (sn * wn + jnp.max(b) + 1.0).reshape(_NB, 1, _BB)

    Wp = jnp.pad(W, ((0, 0), (0, _AP - _A)))
    bp = jnp.pad(b, (0, _AP - _A), constant_values=_NEG)

    actions, log_prob = pl.pallas_call(
        _gfn_kernel,
        grid=(_NB, _NA),
        in_specs=[
            pl.BlockSpec((_BB, _D), lambda i, j: (i, 0)),
            pl.BlockSpec((_D, _TA), lambda i, j: (0, j)),
            pl.BlockSpec((_TA,), lambda i, j: (j,)),
            pl.BlockSpec((1, 1, _BB), lambda i, j: (i, 0, 0)),
            pl.BlockSpec((_BB, _TA), lambda i, j: (i, j)),
        ],
        out_specs=[
            pl.BlockSpec((1, 1, _BB), lambda i, j: (i, 0, 0)),
            pl.BlockSpec((1, 1, _BB), lambda i, j: (i, 0, 0)),
        ],
        out_shape=[
            jax.ShapeDtypeStruct((_NB, 1, _BB), jnp.int32),
            jax.ShapeDtypeStruct((_NB, 1, _BB), jnp.float32),
        ],
        scratch_shapes=[
            pltpu.VMEM((_BB, _CH), jnp.float32),
            pltpu.VMEM((_BB, _CH), jnp.int32),
            pltpu.VMEM((_BB, _CH), jnp.float32),
            pltpu.VMEM((_BB, _CH), jnp.float32),
        ],
        compiler_params=pltpu.CompilerParams(
            dimension_semantics=("parallel", "arbitrary"),
        ),
    )(s, Wp, bp, c, u)
    return (actions.reshape(_B), log_prob.reshape(_B))


# speedup vs baseline: 1.5880x; 1.0668x over previous
import jax
import jax.numpy as jnp
from jax.experimental import pallas as pl
from jax.experimental.pallas import tpu as pltpu

_B = 1024
_D = 16
_A = 100000

_BB = 64
_TA = 12800
_CH = 128
_NC = _TA // _CH
_NB = _B // _BB
_NA = (_A + _TA - 1) // _TA
_AP = _NA * _TA

_NEG = -1e30
_IMAX = 2**31 - 1


def _gfn_kernel(s_ref, w_ref, b_ref, c_ref, u_ref, act_ref, lp_ref,
                az_ref, ac_ref, al_ref, as_ref):
    a = pl.program_id(1)

    @pl.when(a == 0)
    def _init():
        az_ref[:] = jnp.full((_BB, _CH), _NEG, jnp.float32)
        ac_ref[:] = jnp.zeros((_BB, _CH), jnp.int32)
        al_ref[:] = jnp.zeros((_BB, _CH), jnp.float32)
        as_ref[:] = jnp.zeros((_BB, _CH), jnp.float32)

    logits = jnp.dot(s_ref[:], w_ref[:], preferred_element_type=jnp.float32)
    c = c_ref[0, 0, :][:, None]

    acc_z = az_ref[:]
    acc_c = ac_ref[:]
    acc_l = al_ref[:]
    acc_s = as_ref[:]
    for ci in range(_NC):
        sl = slice(ci * _CH, (ci + 1) * _CH)
        lgt = logits[:, sl] + b_ref[0, 0, sl][None, :]
        g = -jnp.log(-jnp.log(u_ref[:, sl]))
        z = lgt + jnp.minimum(g, 100.0)
        upd = z > acc_z
        cid = jnp.full((_BB, _CH), a * _NC + ci, jnp.int32)
        acc_z = jnp.where(upd, z, acc_z)
        acc_c = jnp.where(upd, cid, acc_c)
        acc_l = jnp.where(upd, lgt, acc_l)
        acc_s = acc_s + jnp.exp(lgt - c)
    az_ref[:] = acc_z
    ac_ref[:] = acc_c
    al_ref[:] = acc_l
    as_ref[:] = acc_s

    @pl.when(a == _NA - 1)
    def _fin():
        lane = jax.lax.broadcasted_iota(jnp.int32, (_BB, _CH), 1)
        col = ac_ref[:] * _CH + lane
        acc_z = az_ref[:]
        zrow = jnp.max(acc_z, axis=1)
        eq = acc_z == zrow[:, None]
        idxrow = jnp.min(jnp.where(eq, col, _IMAX), axis=1)
        lrow = jnp.sum(jnp.where(col == idxrow[:, None], al_ref[:], 0.0),
                       axis=1)
        srow = jnp.sum(as_ref[:], axis=1)
        act_ref[0, 0, :] = idxrow
        lp_ref[0, 0, :] = lrow - (c_ref[0, 0, :] + jnp.log(srow))


def kernel(s, u, W, b):
    wn = jnp.sqrt(jnp.max(jnp.sum(W * W, axis=0)))
    sn = jnp.sqrt(jnp.sum(s * s, axis=1))
    c = (sn * wn + jnp.max(b) + 1.0).reshape(_NB, 1, _BB)

    Wp = jnp.pad(W, ((0, 0), (0, _AP - _A)))
    bp = jnp.pad(b, (0, _AP - _A), constant_values=_NEG).reshape(_NA, 1, _TA)

    actions, log_prob = pl.pallas_call(
        _gfn_kernel,
        grid=(_NB, _NA),
        in_specs=[
            pl.BlockSpec((_BB, _D), lambda i, j: (i, 0)),
            pl.BlockSpec((_D, _TA), lambda i, j: (0, j)),
            pl.BlockSpec((1, 1, _TA), lambda i, j: (j, 0, 0)),
            pl.BlockSpec((1, 1, _BB), lambda i, j: (i, 0, 0)),
            pl.BlockSpec((_BB, _TA), lambda i, j: (i, j)),
        ],
        out_specs=[
            pl.BlockSpec((1, 1, _BB), lambda i, j: (i, 0, 0)),
            pl.BlockSpec((1, 1, _BB), lambda i, j: (i, 0, 0)),
        ],
        out_shape=[
            jax.ShapeDtypeStruct((_NB, 1, _BB), jnp.int32),
            jax.ShapeDtypeStruct((_NB, 1, _BB), jnp.float32),
        ],
        scratch_shapes=[
            pltpu.VMEM((_BB, _CH), jnp.float32),
            pltpu.VMEM((_BB, _CH), jnp.int32),
            pltpu.VMEM((_BB, _CH), jnp.float32),
            pltpu.VMEM((_BB, _CH), jnp.float32),
        ],
        compiler_params=pltpu.CompilerParams(
            dimension_semantics=("parallel", "arbitrary"),
        ),
    )(s, Wp, bp, c, u)
    return (actions.reshape(_B), log_prob.reshape(_B))


# speedup vs baseline: 2.0576x; 1.2957x over previous
import jax
import jax.numpy as jnp
from jax.experimental import pallas as pl
from jax.experimental.pallas import tpu as pltpu

_B = 1024
_D = 16
_A = 100000

_BB = 128
_TA = 4096
_CH = 64
_NC = _TA // _CH
_NB = _B // _BB
_NA = (_A + _TA - 1) // _TA
_AP = _NA * _TA

_NEG = -1e30
_IMAX = 2**31 - 1


def _gfn_kernel(s_ref, w_ref, b_ref, c_ref, u_ref, act_ref, lp_ref,
                az_ref, ac_ref, al_ref, as_ref):
    j = pl.program_id(1)

    @pl.when(j == 0)
    def _init():
        az_ref[:] = jnp.full((_CH, _BB), _NEG, jnp.float32)
        ac_ref[:] = jnp.zeros((_CH, _BB), jnp.int32)
        al_ref[:] = jnp.zeros((_CH, _BB), jnp.float32)
        as_ref[:] = jnp.zeros((_CH, _BB), jnp.float32)

    logits = jnp.dot(w_ref[:], s_ref[:], preferred_element_type=jnp.float32)
    c = c_ref[0, 0, :][None, :]

    acc_z = az_ref[:]
    acc_c = ac_ref[:]
    acc_l = al_ref[:]
    acc_s = as_ref[:]
    for ci in range(_NC):
        sl = slice(ci * _CH, (ci + 1) * _CH)
        lgt = logits[sl, :] + b_ref[sl, 0][:, None]
        g = -jnp.log(-jnp.log(u_ref[sl, :]))
        z = lgt + jnp.minimum(g, 100.0)
        upd = z > acc_z
        cid = jnp.full((_CH, _BB), j * _NC + ci, jnp.int32)
        acc_z = jnp.where(upd, z, acc_z)
        acc_c = jnp.where(upd, cid, acc_c)
        acc_l = jnp.where(upd, lgt, acc_l)
        acc_s = acc_s + jnp.exp(lgt - c)
    az_ref[:] = acc_z
    ac_ref[:] = acc_c
    al_ref[:] = acc_l
    as_ref[:] = acc_s

    @pl.when(j == _NA - 1)
    def _fin():
        srow = jax.lax.broadcasted_iota(jnp.int32, (_CH, _BB), 0)
        col = ac_ref[:] * _CH + srow
        acc_z = az_ref[:]
        zmax = jnp.max(acc_z, axis=0)
        eq = acc_z == zmax[None, :]
        idx = jnp.min(jnp.where(eq, col, _IMAX), axis=0)
        lsel = jnp.sum(jnp.where(col == idx[None, :], al_ref[:], 0.0), axis=0)
        ssum = jnp.sum(as_ref[:], axis=0)
        act_ref[0, 0, :] = idx
        lp_ref[0, 0, :] = lsel - (c_ref[0, 0, :] + jnp.log(ssum))


def kernel(s, u, W, b):
    wn = jnp.sqrt(jnp.max(jnp.sum(W * W, axis=0)))
    sn = jnp.sqrt(jnp.sum(s * s, axis=1))
    c = (sn * wn + jnp.max(b) + 1.0).reshape(_NB, 1, _BB)

    uT = u.T
    sT = s.T
    WpT = jnp.pad(W.T, ((0, _AP - _A), (0, 0)))
    bp = jnp.pad(b, (0, _AP - _A), constant_values=_NEG).reshape(_AP, 1)

    actions, log_prob = pl.pallas_call(
        _gfn_kernel,
        grid=(_NB, _NA),
        in_specs=[
            pl.BlockSpec((_D, _BB), lambda i, j: (0, i)),
            pl.BlockSpec((_TA, _D), lambda i, j: (j, 0)),
            pl.BlockSpec((_TA, 1), lambda i, j: (j, 0)),
            pl.BlockSpec((1, 1, _BB), lambda i, j: (i, 0, 0)),
            pl.BlockSpec((_TA, _BB), lambda i, j: (j, i)),
        ],
        out_specs=[
            pl.BlockSpec((1, 1, _BB), lambda i, j: (i, 0, 0)),
            pl.BlockSpec((1, 1, _BB), lambda i, j: (i, 0, 0)),
        ],
        out_shape=[
            jax.ShapeDtypeStruct((_NB, 1, _BB), jnp.int32),
            jax.ShapeDtypeStruct((_NB, 1, _BB), jnp.float32),
        ],
        scratch_shapes=[
            pltpu.VMEM((_CH, _BB), jnp.float32),
            pltpu.VMEM((_CH, _BB), jnp.int32),
            pltpu.VMEM((_CH, _BB), jnp.float32),
            pltpu.VMEM((_CH, _BB), jnp.float32),
        ],
        compiler_params=pltpu.CompilerParams(
            dimension_semantics=("parallel", "arbitrary"),
        ),
    )(sT, WpT, bp, c, uT)
    return (actions.reshape(_B), log_prob.reshape(_B))


# speedup vs baseline: 3.1059x; 1.5095x over previous
import jax
import jax.numpy as jnp
from jax.experimental import pallas as pl
from jax.experimental.pallas import tpu as pltpu

_B = 1024
_D = 16
_A = 100000

_BB = 1024
_TA = 512
_CH = 8
_NC = _TA // _CH
_NA = (_A + _TA - 1) // _TA
_AP = _NA * _TA

_NEG = -1e30
_IMAX = 2**31 - 1


def _gfn_kernel(s_ref, w_ref, b_ref, c_ref, u_ref, act_ref, lp_ref,
                az_ref, ac_ref, al_ref, as_ref):
    j = pl.program_id(0)

    @pl.when(j == 0)
    def _init():
        az_ref[:] = jnp.full((_CH, _BB), _NEG, jnp.float32)
        ac_ref[:] = jnp.zeros((_CH, _BB), jnp.int32)
        al_ref[:] = jnp.zeros((_CH, _BB), jnp.float32)
        as_ref[:] = jnp.zeros((_CH, _BB), jnp.float32)

    logits = jax.lax.dot_general(
        w_ref[:], s_ref[:], (((0,), (0,)), ((), ())),
        preferred_element_type=jnp.float32)
    c = c_ref[0, 0, :][None, :]

    acc_z = az_ref[:]
    acc_c = ac_ref[:]
    acc_l = al_ref[:]
    acc_s = as_ref[:]
    for ci in range(_NC):
        sl = slice(ci * _CH, (ci + 1) * _CH)
        lgt = logits[sl, :] + b_ref[sl, 0][:, None]
        g = -jnp.log(-jnp.log(u_ref[sl, :]))
        z = lgt + jnp.minimum(g, 100.0)
        upd = z > acc_z
        cid = jnp.full((_CH, _BB), j * _NC + ci, jnp.int32)
        acc_z = jnp.where(upd, z, acc_z)
        acc_c = jnp.where(upd, cid, acc_c)
        acc_l = jnp.where(upd, lgt, acc_l)
        acc_s = acc_s + jnp.exp(lgt - c)
    az_ref[:] = acc_z
    ac_ref[:] = acc_c
    al_ref[:] = acc_l
    as_ref[:] = acc_s

    @pl.when(j == _NA - 1)
    def _fin():
        srow = jax.lax.broadcasted_iota(jnp.int32, (_CH, _BB), 0)
        col = ac_ref[:] * _CH + srow
        acc_z = az_ref[:]
        zmax = jnp.max(acc_z, axis=0)
        eq = acc_z == zmax[None, :]
        idx = jnp.min(jnp.where(eq, col, _IMAX), axis=0)
        lsel = jnp.sum(jnp.where(col == idx[None, :], al_ref[:], 0.0), axis=0)
        ssum = jnp.sum(as_ref[:], axis=0)
        act_ref[0, 0, :] = idx
        lp_ref[0, 0, :] = lsel - (c_ref[0, 0, :] + jnp.log(ssum))


def kernel(s, u, W, b):
    wn = jnp.sqrt(jnp.max(jnp.sum(W * W, axis=0)))
    sn = jnp.sqrt(jnp.sum(s * s, axis=1))
    c = (sn * wn + jnp.max(b) + 1.0).reshape(1, 1, _BB)

    uT = u.T
    sT = s.T
    Wp = jnp.pad(W, ((0, 0), (0, _AP - _A)))
    bp = jnp.pad(b, (0, _AP - _A), constant_values=_NEG).reshape(_AP, 1)

    actions, log_prob = pl.pallas_call(
        _gfn_kernel,
        grid=(_NA,),
        in_specs=[
            pl.BlockSpec((_D, _BB), lambda j: (0, 0)),
            pl.BlockSpec((_D, _TA), lambda j: (0, j)),
            pl.BlockSpec((_TA, 1), lambda j: (j, 0)),
            pl.BlockSpec((1, 1, _BB), lambda j: (0, 0, 0)),
            pl.BlockSpec((_TA, _BB), lambda j: (j, 0)),
        ],
        out_specs=[
            pl.BlockSpec((1, 1, _BB), lambda j: (0, 0, 0)),
            pl.BlockSpec((1, 1, _BB), lambda j: (0, 0, 0)),
        ],
        out_shape=[
            jax.ShapeDtypeStruct((1, 1, _BB), jnp.int32),
            jax.ShapeDtypeStruct((1, 1, _BB), jnp.float32),
        ],
        scratch_shapes=[
            pltpu.VMEM((_CH, _BB), jnp.float32),
            pltpu.VMEM((_CH, _BB), jnp.int32),
            pltpu.VMEM((_CH, _BB), jnp.float32),
            pltpu.VMEM((_CH, _BB), jnp.float32),
        ],
        compiler_params=pltpu.CompilerParams(
            dimension_semantics=("arbitrary",),
        ),
    )(sT, Wp, bp, c, uT)
    return (actions.reshape(_B), log_prob.reshape(_B))


# speedup vs baseline: 3.2611x; 1.0500x over previous
import jax
import jax.numpy as jnp
from jax.experimental import pallas as pl
from jax.experimental.pallas import tpu as pltpu

_B = 1024
_D = 16
_A = 100000

_BB = 1024
_TA = 1024
_CH = 8
_NC = _TA // _CH
_NA = (_A + _TA - 1) // _TA
_AP = _NA * _TA

_NEG = -1e30
_IMAX = 2**31 - 1
_NLN2 = -0.6931471805599453
_TINY = 1.1754943508222875e-38


def _gfn_kernel(s_ref, w_ref, b_ref, c_ref, u_ref, act_ref, lp_ref,
                az_ref, ac_ref, al_ref, as_ref):
    j = pl.program_id(0)

    @pl.when(j == 0)
    def _init():
        az_ref[:] = jnp.full((_CH, _BB), _NEG, jnp.float32)
        ac_ref[:] = jnp.zeros((_CH, _BB), jnp.int32)
        al_ref[:] = jnp.zeros((_CH, _BB), jnp.float32)
        as_ref[:] = jnp.zeros((_CH, _BB), jnp.float32)

    logits = jax.lax.dot_general(
        w_ref[:], s_ref[:], (((0,), (0,)), ((), ())),
        preferred_element_type=jnp.float32)
    c = c_ref[0, 0, :][None, :]

    acc_z = az_ref[:]
    acc_c = ac_ref[:]
    acc_l = al_ref[:]
    acc_s = as_ref[:]
    for ci in range(_NC):
        sl = slice(ci * _CH, (ci + 1) * _CH)
        lgt = logits[sl, :] + b_ref[sl, 0][:, None]
        neg_log_u = jnp.log2(u_ref[sl, :]) * _NLN2
        g = jnp.log2(jnp.maximum(neg_log_u, _TINY)) * _NLN2
        z = lgt + g
        upd = z > acc_z
        cid = jnp.full((_CH, _BB), j * _NC + ci, jnp.int32)
        acc_z = jnp.where(upd, z, acc_z)
        acc_c = jnp.where(upd, cid, acc_c)
        acc_l = jnp.where(upd, lgt, acc_l)
        acc_s = acc_s + jnp.exp(lgt - c)
    az_ref[:] = acc_z
    ac_ref[:] = acc_c
    al_ref[:] = acc_l
    as_ref[:] = acc_s

    @pl.when(j == _NA - 1)
    def _fin():
        srow = jax.lax.broadcasted_iota(jnp.int32, (_CH, _BB), 0)
        col = ac_ref[:] * _CH + srow
        acc_z = az_ref[:]
        zmax = jnp.max(acc_z, axis=0)
        eq = acc_z == zmax[None, :]
        idx = jnp.min(jnp.where(eq, col, _IMAX), axis=0)
        lsel = jnp.sum(jnp.where(col == idx[None, :], al_ref[:], 0.0), axis=0)
        ssum = jnp.sum(as_ref[:], axis=0)
        act_ref[0, 0, :] = idx
        lp_ref[0, 0, :] = lsel - (c_ref[0, 0, :] + jnp.log(ssum))


def kernel(s, u, W, b):
    wn = jnp.sqrt(jnp.max(jnp.sum(W * W, axis=0)))
    sn = jnp.sqrt(jnp.sum(s * s, axis=1))
    c = (sn * wn + jnp.max(b) + 1.0).reshape(1, 1, _BB)

    uT = u.T
    sT = s.T
    Wp = jnp.pad(W, ((0, 0), (0, _AP - _A)))
    bp = jnp.pad(b, (0, _AP - _A), constant_values=_NEG).reshape(_AP, 1)

    actions, log_prob = pl.pallas_call(
        _gfn_kernel,
        grid=(_NA,),
        in_specs=[
            pl.BlockSpec((_D, _BB), lambda j: (0, 0)),
            pl.BlockSpec((_D, _TA), lambda j: (0, j)),
            pl.BlockSpec((_TA, 1), lambda j: (j, 0)),
            pl.BlockSpec((1, 1, _BB), lambda j: (0, 0, 0)),
            pl.BlockSpec((_TA, _BB), lambda j: (j, 0)),
        ],
        out_specs=[
            pl.BlockSpec((1, 1, _BB), lambda j: (0, 0, 0)),
            pl.BlockSpec((1, 1, _BB), lambda j: (0, 0, 0)),
        ],
        out_shape=[
            jax.ShapeDtypeStruct((1, 1, _BB), jnp.int32),
            jax.ShapeDtypeStruct((1, 1, _BB), jnp.float32),
        ],
        scratch_shapes=[
            pltpu.VMEM((_CH, _BB), jnp.float32),
            pltpu.VMEM((_CH, _BB), jnp.int32),
            pltpu.VMEM((_CH, _BB), jnp.float32),
            pltpu.VMEM((_CH, _BB), jnp.float32),
        ],
        compiler_params=pltpu.CompilerParams(
            dimension_semantics=("arbitrary",),
        ),
    )(sT, Wp, bp, c, uT)
    return (actions.reshape(_B), log_prob.reshape(_B))


# speedup vs baseline: 3.7434x; 1.1479x over previous
import jax
import jax.numpy as jnp
from jax.experimental import pallas as pl
from jax.experimental.pallas import tpu as pltpu

_B = 1024
_D = 16
_A = 100000

_BB = 1024
_TA = 1024
_CH = 8
_NC = _TA // _CH
_NA = (_A + _TA - 1) // _TA
_AP = _NA * _TA

_NEG = -1e30
_IMAX = 2**31 - 1
_NLN2 = -0.6931471805599453
_TINY = 1.1754943508222875e-38


def _gfn_kernel(s_ref, w_ref, b_ref, c_ref, u_ref, act_ref, lp_ref,
                az_ref, ac_ref, al_ref, as_ref):
    j = pl.program_id(0)

    @pl.when(j == 0)
    def _init():
        az_ref[:] = jnp.full((_CH, _BB), _NEG, jnp.float32)
        ac_ref[:] = jnp.zeros((_CH, _BB), jnp.int32)
        al_ref[:] = jnp.zeros((_CH, _BB), jnp.float32)
        as_ref[:] = jnp.zeros((_CH, _BB), jnp.float32)

    logits = jax.lax.dot_general(
        w_ref[:], s_ref[:], (((0,), (0,)), ((), ())),
        preferred_element_type=jnp.float32)
    c = c_ref[0, 0, :][None, :]

    acc_z = az_ref[:]
    acc_c = ac_ref[:]
    acc_l = al_ref[:]
    acc_s = as_ref[:]
    for ci in range(_NC):
        sl = slice(ci * _CH, (ci + 1) * _CH)
        lgt = logits[sl, :] + b_ref[sl, 0][:, None]
        e = jnp.maximum(-jnp.log(u_ref[sl, :]), _TINY)
        z = lgt - jnp.log(e)
        upd = z > acc_z
        cid = jnp.full((_CH, _BB), j * _NC + ci, jnp.int32)
        acc_z = jnp.where(upd, z, acc_z)
        acc_c = jnp.where(upd, cid, acc_c)
        acc_l = jnp.where(upd, lgt, acc_l)
        acc_s = acc_s + jnp.exp(lgt - c)
    az_ref[:] = acc_z
    ac_ref[:] = acc_c
    al_ref[:] = acc_l
    as_ref[:] = acc_s

    @pl.when(j == _NA - 1)
    def _fin():
        srow = jax.lax.broadcasted_iota(jnp.int32, (_CH, _BB), 0)
        col = ac_ref[:] * _CH + srow
        acc_z = az_ref[:]
        zmax = jnp.max(acc_z, axis=0)
        eq = acc_z == zmax[None, :]
        idx = jnp.min(jnp.where(eq, col, _IMAX), axis=0)
        lsel = jnp.sum(jnp.where(col == idx[None, :], al_ref[:], 0.0), axis=0)
        ssum = jnp.sum(as_ref[:], axis=0)
        act_ref[0, 0, :] = idx
        lp_ref[0, 0, :] = lsel - (c_ref[0, 0, :] + jnp.log(ssum))


def kernel(s, u, W, b):
    wn = jnp.sqrt(jnp.max(jnp.sum(W * W, axis=0)))
    sn = jnp.sqrt(jnp.sum(s * s, axis=1))
    c = (sn * wn + jnp.max(b) + 1.0).reshape(1, 1, _BB)

    uT = u.T
    sT = s.T
    Wp = jnp.pad(W, ((0, 0), (0, _AP - _A)))
    bp = jnp.pad(b, (0, _AP - _A), constant_values=_NEG).reshape(_AP, 1)

    actions, log_prob = pl.pallas_call(
        _gfn_kernel,
        grid=(_NA,),
        in_specs=[
            pl.BlockSpec((_D, _BB), lambda j: (0, 0)),
            pl.BlockSpec((_D, _TA), lambda j: (0, j)),
            pl.BlockSpec((_TA, 1), lambda j: (j, 0)),
            pl.BlockSpec((1, 1, _BB), lambda j: (0, 0, 0)),
            pl.BlockSpec((_TA, _BB), lambda j: (j, 0)),
        ],
        out_specs=[
            pl.BlockSpec((1, 1, _BB), lambda j: (0, 0, 0)),
            pl.BlockSpec((1, 1, _BB), lambda j: (0, 0, 0)),
        ],
        out_shape=[
            jax.ShapeDtypeStruct((1, 1, _BB), jnp.int32),
            jax.ShapeDtypeStruct((1, 1, _BB), jnp.float32),
        ],
        scratch_shapes=[
            pltpu.VMEM((_CH, _BB), jnp.float32),
            pltpu.VMEM((_CH, _BB), jnp.int32),
            pltpu.VMEM((_CH, _BB), jnp.float32),
            pltpu.VMEM((_CH, _BB), jnp.float32),
        ],
        compiler_params=pltpu.CompilerParams(
            dimension_semantics=("arbitrary",),
        ),
    )(sT, Wp, bp, c, uT)
    return (actions.reshape(_B), log_prob.reshape(_B))


# speedup vs baseline: 3.7948x; 1.0137x over previous
import jax
import jax.numpy as jnp
from jax.experimental import pallas as pl
from jax.experimental.pallas import tpu as pltpu

_B = 1024
_D = 16
_A = 100000

_BB = 1024
_TA = 2048
_CH = 8
_NC = _TA // _CH
_NA = (_A + _TA - 1) // _TA
_AP = _NA * _TA

_NEG = -1e30
_IMAX = 2**31 - 1
_NLN2 = -0.6931471805599453
_TINY = 1.1754943508222875e-38


def _gfn_kernel(s_ref, w_ref, b_ref, c_ref, u_ref, act_ref, lp_ref,
                az_ref, ac_ref, al_ref, as_ref):
    j = pl.program_id(0)

    @pl.when(j == 0)
    def _init():
        az_ref[:] = jnp.full((_CH, _BB), _NEG, jnp.float32)
        ac_ref[:] = jnp.zeros((_CH, _BB), jnp.int32)
        al_ref[:] = jnp.zeros((_CH, _BB), jnp.float32)
        as_ref[:] = jnp.zeros((_CH, _BB), jnp.float32)

    logits = jax.lax.dot_general(
        w_ref[:], s_ref[:], (((0,), (0,)), ((), ())),
        preferred_element_type=jnp.float32)
    c = c_ref[0, 0, :][None, :]

    acc_z = az_ref[:]
    acc_c = ac_ref[:]
    acc_l = al_ref[:]
    acc_s = as_ref[:]
    for ci in range(_NC):
        sl = slice(ci * _CH, (ci + 1) * _CH)
        lgt = logits[sl, :] + b_ref[sl, 0][:, None]
        e = jnp.maximum(-jnp.log(u_ref[sl, :]), _TINY)
        z = lgt - jnp.log(e)
        upd = z > acc_z
        cid = jnp.full((_CH, _BB), j * _NC + ci, jnp.int32)
        acc_z = jnp.where(upd, z, acc_z)
        acc_c = jnp.where(upd, cid, acc_c)
        acc_l = jnp.where(upd, lgt, acc_l)
        acc_s = acc_s + jnp.exp(lgt - c)
    az_ref[:] = acc_z
    ac_ref[:] = acc_c
    al_ref[:] = acc_l
    as_ref[:] = acc_s

    @pl.when(j == _NA - 1)
    def _fin():
        srow = jax.lax.broadcasted_iota(jnp.int32, (_CH, _BB), 0)
        col = ac_ref[:] * _CH + srow
        acc_z = az_ref[:]
        zmax = jnp.max(acc_z, axis=0)
        eq = acc_z == zmax[None, :]
        idx = jnp.min(jnp.where(eq, col, _IMAX), axis=0)
        lsel = jnp.sum(jnp.where(col == idx[None, :], al_ref[:], 0.0), axis=0)
        ssum = jnp.sum(as_ref[:], axis=0)
        act_ref[0, 0, :] = idx
        lp_ref[0, 0, :] = lsel - (c_ref[0, 0, :] + jnp.log(ssum))


def kernel(s, u, W, b):
    wn = jnp.sqrt(jnp.max(jnp.sum(W * W, axis=0)))
    sn = jnp.sqrt(jnp.sum(s * s, axis=1))
    c = (sn * wn + jnp.max(b) + 1.0).reshape(1, 1, _BB)

    uT = u.T
    sT = s.T
    Wp = jnp.pad(W, ((0, 0), (0, _AP - _A)))
    bp = jnp.pad(b, (0, _AP - _A), constant_values=_NEG).reshape(_AP, 1)

    actions, log_prob = pl.pallas_call(
        _gfn_kernel,
        grid=(_NA,),
        in_specs=[
            pl.BlockSpec((_D, _BB), lambda j: (0, 0)),
            pl.BlockSpec((_D, _TA), lambda j: (0, j)),
            pl.BlockSpec((_TA, 1), lambda j: (j, 0)),
            pl.BlockSpec((1, 1, _BB), lambda j: (0, 0, 0)),
            pl.BlockSpec((_TA, _BB), lambda j: (j, 0)),
        ],
        out_specs=[
            pl.BlockSpec((1, 1, _BB), lambda j: (0, 0, 0)),
            pl.BlockSpec((1, 1, _BB), lambda j: (0, 0, 0)),
        ],
        out_shape=[
            jax.ShapeDtypeStruct((1, 1, _BB), jnp.int32),
            jax.ShapeDtypeStruct((1, 1, _BB), jnp.float32),
        ],
        scratch_shapes=[
            pltpu.VMEM((_CH, _BB), jnp.float32),
            pltpu.VMEM((_CH, _BB), jnp.int32),
            pltpu.VMEM((_CH, _BB), jnp.float32),
            pltpu.VMEM((_CH, _BB), jnp.float32),
        ],
        compiler_params=pltpu.CompilerParams(
            dimension_semantics=("arbitrary",),
        ),
    )(sT, Wp, bp, c, uT)
    return (actions.reshape(_B), log_prob.reshape(_B))


# speedup vs baseline: 3.7986x; 1.0010x over previous
import jax
import jax.numpy as jnp
from jax.experimental import pallas as pl
from jax.experimental.pallas import tpu as pltpu

_B = 1024
_D = 16
_A = 100000

_BB = 1024
_TA = 2048
_CH = 8
_NC = _TA // _CH
_NA = (_A + _TA - 1) // _TA
_AP = _NA * _TA

_NEG = -1e30
_IMAX = 2**31 - 1
_TINY = 1.1754943508222875e-38


def _gfn_kernel(s_ref, w_ref, b_ref, c_ref, u_ref, act_ref, lp_ref,
                az_ref, ac_ref, al_ref, as_ref):
    j = pl.program_id(0)

    @pl.when(j == 0)
    def _init():
        az_ref[:] = jnp.full((_CH, _BB), _NEG, jnp.float32)
        ac_ref[:] = jnp.zeros((_CH, _BB), jnp.int32)
        al_ref[:] = jnp.zeros((_CH, _BB), jnp.float32)
        as_ref[:] = jnp.zeros((_CH, _BB), jnp.float32)

    logits = jax.lax.dot_general(
        w_ref[:], s_ref[:], (((0,), (0,)), ((), ())),
        preferred_element_type=jnp.float32)
    c = c_ref[0, 0, :][None, :]

    acc_z = az_ref[:]
    acc_c = ac_ref[:]
    acc_l = al_ref[:]
    acc_s = as_ref[:]
    for ci in range(_NC):
        sl = slice(ci * _CH, (ci + 1) * _CH)
        lgt = logits[sl, :] + b_ref[sl, 0][:, None]
        e = jnp.maximum(-jnp.log(u_ref[sl, :]), _TINY)
        z = lgt - jnp.log(e)
        upd = z > acc_z
        cid = jnp.full((_CH, _BB), j * _NC + ci, jnp.int32)
        acc_z = jnp.where(upd, z, acc_z)
        acc_c = jnp.where(upd, cid, acc_c)
        acc_l = jnp.where(upd, lgt, acc_l)
        acc_s = acc_s + jnp.exp(lgt - c)
    az_ref[:] = acc_z
    ac_ref[:] = acc_c
    al_ref[:] = acc_l
    as_ref[:] = acc_s

    @pl.when(j == _NA - 1)
    def _fin():
        srow = jax.lax.broadcasted_iota(jnp.int32, (_CH, _BB), 0)
        col = ac_ref[:] * _CH + srow
        acc_z = az_ref[:]
        zmax = jnp.max(acc_z, axis=0)
        eq = acc_z == zmax[None, :]
        idx = jnp.min(jnp.where(eq, col, _IMAX), axis=0)
        lsel = jnp.sum(jnp.where(col == idx[None, :], al_ref[:], 0.0), axis=0)
        ssum = jnp.sum(as_ref[:], axis=0)
        act_ref[0, 0, :] = idx
        lp_ref[0, 0, :] = lsel - (c_ref[0, 0, :] + jnp.log(ssum))


def kernel(s, u, W, b):
    wn = jnp.sqrt(jnp.max(jnp.sum(W * W, axis=0)))
    sn = jnp.sqrt(jnp.sum(s * s, axis=1))
    c = (sn * wn + jnp.max(b) + 1.0).reshape(1, 1, _BB)

    uT = u.T
    sT = s.T
    Wp = jnp.pad(W, ((0, 0), (0, _AP - _A)))
    bp = jnp.pad(b, (0, _AP - _A), constant_values=_NEG).reshape(_AP, 1)

    actions, log_prob = pl.pallas_call(
        _gfn_kernel,
        grid=(_NA,),
        in_specs=[
            pl.BlockSpec((_D, _BB), lambda j: (0, 0)),
            pl.BlockSpec((_D, _TA), lambda j: (0, j)),
            pl.BlockSpec((_TA, 1), lambda j: (j, 0)),
            pl.BlockSpec((1, 1, _BB), lambda j: (0, 0, 0)),
            pl.BlockSpec((_TA, _BB), lambda j: (j, 0)),
        ],
        out_specs=[
            pl.BlockSpec((1, 1, _BB), lambda j: (0, 0, 0)),
            pl.BlockSpec((1, 1, _BB), lambda j: (0, 0, 0)),
        ],
        out_shape=[
            jax.ShapeDtypeStruct((1, 1, _BB), jnp.int32),
            jax.ShapeDtypeStruct((1, 1, _BB), jnp.float32),
        ],
        scratch_shapes=[
            pltpu.VMEM((_CH, _BB), jnp.float32),
            pltpu.VMEM((_CH, _BB), jnp.int32),
            pltpu.VMEM((_CH, _BB), jnp.float32),
            pltpu.VMEM((_CH, _BB), jnp.float32),
        ],
        compiler_params=pltpu.CompilerParams(
            dimension_semantics=("arbitrary",),
        ),
    )(sT, Wp, bp, c, uT)
    return (actions.reshape(_B), log_prob.reshape(_B))
